# Initial kernel scaffold; baseline (speedup 1.0000x reference)
import jax
import jax.numpy as jnp
from jax import lax
from jax.experimental import pallas as pl
from jax.experimental.pallas import tpu as pltpu
from jax.experimental.pallas import tpu_sc as plsc

N_NODES = 50000
N_EDGES = 800000
D_FEAT = 64
DC = 16
NUM_CHUNKS = D_FEAT // DC
NUM_SUBCORES = 16
EDGES_PER_TILE = N_EDGES // NUM_SUBCORES
B = 80
NUM_BATCHES = EDGES_PER_TILE // B
ZROWS = 125
ROWS_PER_TILE = N_NODES // NUM_SUBCORES
NZ = ROWS_PER_TILE // ZROWS


def _sc_spmm_kernel(ebs_cat, rows_hbm, cols_hbm, vl_hbm, vli_hbm, out_hbm,
                    cols_v, rows_v, vl_v, vli_v, g_v, staged, zero_v,
                    acc, sem):
    c = lax.axis_index("c")
    s = lax.axis_index("s")
    row_base = s * ROWS_PER_TILE

    z16 = jnp.zeros((16,), jnp.float32)

    def zbody(i, carry):
        zero_v[i, pl.ds(0, 16)] = z16
        zero_v[i, pl.ds(16, 16)] = z16
        return carry
    lax.fori_loop(0, ZROWS, zbody, 0)

    for j in range(2):
        k = 2 * c + j

        def zero_body(i, carry):
            pltpu.sync_copy(zero_v, acc.at[pl.ds(row_base + i * ZROWS, ZROWS)])
            return carry
        lax.fori_loop(0, NZ, zero_body, 0)
        plsc.subcore_barrier()

        col_off = k * N_NODES

        def batch_body(b, carry):
            eb = s * EDGES_PER_TILE + b * B
            pltpu.sync_copy(cols_hbm.at[pl.ds(eb, B)], cols_v)
            pltpu.sync_copy(rows_hbm.at[pl.ds(eb, B)], rows_v)
            pltpu.sync_copy(vl_hbm.at[pl.ds(eb, B)], vl_v)
            pltpu.sync_copy(vli_hbm.at[pl.ds(eb, B)], vli_v)
            for t in range(B // 16):
                cols_v[pl.ds(t * 16, 16)] = cols_v[pl.ds(t * 16, 16)] + col_off
            pltpu.async_copy(ebs_cat.at[cols_v], g_v, sem).wait()
            for e in range(B):
                idx16 = jnp.full((16,), e, jnp.int32)
                bvli = plsc.load_gather(vli_v, [idx16])
                bvl = plsc.load_gather(vl_v, [idx16])
                g = g_v[e, :]
                staged[e, pl.ds(0, 16)] = bvli * g
                staged[e, pl.ds(16, 16)] = bvl * g
            pltpu.sync_copy(staged, acc.at[rows_v], add=True)
            return carry
        lax.fori_loop(0, NUM_BATCHES, batch_body, 0)
        plsc.subcore_barrier()

        def dump_body(i, carry):
            r0 = row_base + i * ZROWS
            pltpu.sync_copy(acc.at[pl.ds(r0, ZROWS)],
                            out_hbm.at[k, pl.ds(r0, ZROWS)])
            return carry
        lax.fori_loop(0, NZ, dump_body, 0)
        plsc.subcore_barrier()


def _sc_spmm(ebs_cat, rows, cols, vals_L, vals_LI):
    mesh = plsc.VectorSubcoreMesh(core_axis_name="c", subcore_axis_name="s")
    f = pl.kernel(
        _sc_spmm_kernel,
        out_type=jax.ShapeDtypeStruct((NUM_CHUNKS, N_NODES, 2 * DC),
                                      jnp.float32),
        mesh=mesh,
        scratch_types=[
            pltpu.VMEM((B,), jnp.int32),
            pltpu.VMEM((B,), jnp.int32),
            pltpu.VMEM((B,), jnp.float32),
            pltpu.VMEM((B,), jnp.float32),
            pltpu.VMEM((B, DC), jnp.float32),
            pltpu.VMEM((B, 2 * DC), jnp.float32),
            pltpu.VMEM((ZROWS, 2 * DC), jnp.float32),
            pltpu.VMEM_SHARED((N_NODES, 2 * DC), jnp.float32),
            pltpu.SemaphoreType.DMA,
        ],
    )
    return f(ebs_cat, rows, cols, vals_L, vals_LI)


def _combine_kernel(planes_ref, ent_ref, ws_ref, wd_ref, out_ref):
    p = planes_ref[...]
    li = jnp.concatenate([p[i, :, 0:DC] for i in range(NUM_CHUNKS)], axis=1)
    l_ = jnp.concatenate([p[i, :, DC:2 * DC] for i in range(NUM_CHUNKS)],
                         axis=1)
    acc = jnp.dot(li, ws_ref[...], preferred_element_type=jnp.float32)
    acc += jnp.dot(l_ * ent_ref[...], wd_ref[...],
                   preferred_element_type=jnp.float32)
    out_ref[...] = jnp.where(acc >= 0, acc, 0.2 * acc)


def _combine_tc(planes, entity_ebs, W_side, W_dot):
    BN = 400
    grid = (N_NODES // BN,)
    return pl.pallas_call(
        _combine_kernel,
        grid=grid,
        in_specs=[
            pl.BlockSpec((NUM_CHUNKS, BN, 2 * DC), lambda i: (0, i, 0)),
            pl.BlockSpec((BN, D_FEAT), lambda i: (i, 0)),
            pl.BlockSpec((D_FEAT, D_FEAT), lambda i: (0, 0)),
            pl.BlockSpec((D_FEAT, D_FEAT), lambda i: (0, 0)),
        ],
        out_specs=pl.BlockSpec((BN, D_FEAT), lambda i: (i, 0)),
        out_shape=jax.ShapeDtypeStruct((N_NODES, D_FEAT), jnp.float32),
    )(planes, entity_ebs, W_side, W_dot)


@jax.jit
def kernel(ebs, entity_ebs, vals_L, vals_LI, W_side, W_dot, edge_index):
    rows = edge_index[0]
    cols = edge_index[1]
    ebs_cat = jnp.concatenate(
        [ebs[:, k * DC:(k + 1) * DC] for k in range(NUM_CHUNKS)], axis=0)
    planes = _sc_spmm(ebs_cat, rows, cols, vals_L, vals_LI)
    return _combine_tc(planes, entity_ebs, W_side, W_dot)


# speedup vs baseline: 1.5335x; 1.5335x over previous
import jax
import jax.numpy as jnp
from jax import lax
from jax.experimental import pallas as pl
from jax.experimental.pallas import tpu as pltpu
from jax.experimental.pallas import tpu_sc as plsc

N_NODES = 50000
N_PAD = 50176
N_EDGES = 800000
D_FEAT = 64
DC = 16
NUM_CHUNKS = D_FEAT // DC
NUM_SUBCORES = 16
EDGES_PER_TILE = N_EDGES // NUM_SUBCORES
B = 80
NUM_BATCHES = EDGES_PER_TILE // B
ZROWS = 392
ROWS_PER_TILE = N_PAD // NUM_SUBCORES
NZ = ROWS_PER_TILE // ZROWS


def _sc_spmm_kernel(ebs_cat, rows_hbm, cols_hbm, vl_hbm, vli_hbm, out_hbm,
                    cols_v, rows_v, vl_v, vli_v, g_v, staged, zero_v,
                    acc, sem):
    c = lax.axis_index("c")
    s = lax.axis_index("s")
    row_base = s * ROWS_PER_TILE

    z16 = jnp.zeros((16,), jnp.float32)

    def zbody(i, carry):
        zero_v[i, pl.ds(0, 16)] = z16
        zero_v[i, pl.ds(16, 16)] = z16
        return carry
    lax.fori_loop(0, ZROWS, zbody, 0)

    for j in range(2):
        k = 2 * c + j

        def zero_body(i, carry):
            pltpu.sync_copy(zero_v, acc.at[pl.ds(row_base + i * ZROWS, ZROWS)])
            return carry
        lax.fori_loop(0, NZ, zero_body, 0)
        plsc.subcore_barrier()

        col_off = k * N_NODES

        def batch_body(b, carry):
            eb = s * EDGES_PER_TILE + b * B
            pltpu.sync_copy(cols_hbm.at[pl.ds(eb, B)], cols_v)
            pltpu.sync_copy(rows_hbm.at[pl.ds(eb, B)], rows_v)
            pltpu.sync_copy(vl_hbm.at[pl.ds(eb, B)], vl_v)
            pltpu.sync_copy(vli_hbm.at[pl.ds(eb, B)], vli_v)
            for t in range(B // 16):
                cols_v[pl.ds(t * 16, 16)] = cols_v[pl.ds(t * 16, 16)] + col_off
            pltpu.async_copy(ebs_cat.at[cols_v], g_v, sem).wait()
            for e in range(B):
                idx16 = jnp.full((16,), e, jnp.int32)
                bvli = plsc.load_gather(vli_v, [idx16])
                bvl = plsc.load_gather(vl_v, [idx16])
                g = g_v[e, :]
                staged[e, pl.ds(0, 16)] = bvli * g
                staged[e, pl.ds(16, 16)] = bvl * g
            pltpu.sync_copy(staged, acc.at[rows_v], add=True)
            return carry
        lax.fori_loop(0, NUM_BATCHES, batch_body, 0)
        plsc.subcore_barrier()

        def dump_body(i, carry):
            r0 = row_base + i * ZROWS
            pltpu.sync_copy(acc.at[pl.ds(r0, ZROWS)],
                            out_hbm.at[k, pl.ds(r0, ZROWS)])
            return carry
        lax.fori_loop(0, NZ, dump_body, 0)
        plsc.subcore_barrier()


def _sc_spmm(ebs_cat, rows, cols, vals_L, vals_LI):
    mesh = plsc.VectorSubcoreMesh(core_axis_name="c", subcore_axis_name="s")
    f = pl.kernel(
        _sc_spmm_kernel,
        out_type=jax.ShapeDtypeStruct((NUM_CHUNKS, N_PAD, 2 * DC),
                                      jnp.float32),
        mesh=mesh,
        compiler_params=pltpu.CompilerParams(
            needs_layout_passes=False, use_tc_tiling_on_sc=False),
        scratch_types=[
            pltpu.VMEM((B,), jnp.int32),
            pltpu.VMEM((B,), jnp.int32),
            pltpu.VMEM((B,), jnp.float32),
            pltpu.VMEM((B,), jnp.float32),
            pltpu.VMEM((B, DC), jnp.float32),
            pltpu.VMEM((B, 2 * DC), jnp.float32),
            pltpu.VMEM((ZROWS, 2 * DC), jnp.float32),
            pltpu.VMEM_SHARED((N_PAD, 2 * DC), jnp.float32),
            pltpu.SemaphoreType.DMA,
        ],
    )
    return f(ebs_cat, rows, cols, vals_L, vals_LI)


def _combine_kernel(planes_ref, ent_ref, ws_ref, wd_ref, out_ref):
    p = planes_ref[...]
    li = jnp.concatenate([p[i, :, 0:DC] for i in range(NUM_CHUNKS)], axis=1)
    l_ = jnp.concatenate([p[i, :, DC:2 * DC] for i in range(NUM_CHUNKS)],
                         axis=1)
    acc = jnp.dot(li, ws_ref[...], preferred_element_type=jnp.float32)
    acc += jnp.dot(l_ * ent_ref[...], wd_ref[...],
                   preferred_element_type=jnp.float32)
    out_ref[...] = jnp.where(acc >= 0, acc, 0.2 * acc)


def _combine_tc(planes, entity_ebs, W_side, W_dot):
    BN = 400
    grid = (N_NODES // BN,)
    return pl.pallas_call(
        _combine_kernel,
        grid=grid,
        in_specs=[
            pl.BlockSpec((NUM_CHUNKS, BN, 2 * DC), lambda i: (0, i, 0)),
            pl.BlockSpec((BN, D_FEAT), lambda i: (i, 0)),
            pl.BlockSpec((D_FEAT, D_FEAT), lambda i: (0, 0)),
            pl.BlockSpec((D_FEAT, D_FEAT), lambda i: (0, 0)),
        ],
        out_specs=pl.BlockSpec((BN, D_FEAT), lambda i: (i, 0)),
        out_shape=jax.ShapeDtypeStruct((N_NODES, D_FEAT), jnp.float32),
    )(planes, entity_ebs, W_side, W_dot)


@jax.jit
def kernel(ebs, entity_ebs, vals_L, vals_LI, W_side, W_dot, edge_index):
    rows = edge_index[0]
    cols = edge_index[1]
    ebs_cat = jnp.concatenate(
        [ebs[:, k * DC:(k + 1) * DC] for k in range(NUM_CHUNKS)], axis=0)
    planes = _sc_spmm(ebs_cat, rows, cols, vals_L, vals_LI)
    return _combine_tc(planes, entity_ebs, W_side, W_dot)
